# baseline (speedup 1.0000x reference)
import jax
import jax.numpy as jnp
from jax import lax
from jax.experimental import pallas as pl
from jax.experimental.pallas import tpu as pltpu
from jax.experimental.pallas import tpu_sc as plsc

N_ROWS = 50
D = 1024
DW = D // 2
B = 16384

_info = plsc.get_sparse_core_info()
_NC, _NS, _L = _info.num_cores, _info.num_subcores, _info.num_lanes
_NW = _NC * _NS
_BPW = B // _NW
_CH = 128
_NCH = _BPW // _CH
_RCH = 64
_NRCH = _BPW // _RCH
_NB = 3


def _body(sigma_hbm, table_hbm, lut_hbm, out_hbm, sigma_v, bits_v, idx_v,
          rows_v, sem_lut, sem_g, sem_s):
    wid = lax.axis_index("s") * _NC + lax.axis_index("c")
    base = pl.multiple_of(wid * _BPW, _BPW)

    pltpu.sync_copy(
        sigma_hbm.at[pl.ds(pl.multiple_of(wid * (_BPW // 2), _BPW // 2),
                           _BPW // 2)], sigma_v)

    iota = lax.iota(jnp.int32, _L)
    for i in range(_BPW // (2 * _L)):
        w = sigma_v[pl.ds(i * _L, _L)]
        lo = lax.bitwise_and(w, jnp.int32(0xFFFF))
        hi = lax.shift_right_logical(w, jnp.int32(16))
        even = iota * jnp.int32(2) + jnp.int32(i * 2 * _L)
        plsc.store_scatter(bits_v, [even], lo)
        plsc.store_scatter(bits_v, [even + jnp.int32(1)], hi)

    lut_cps = [
        pltpu.async_copy(lut_hbm.at[bits_v.at[pl.ds(c * _CH, _CH)]],
                         idx_v.at[pl.ds(c * _CH, _CH)], sem_lut)
        for c in range(_NCH)
    ]
    for cp in lut_cps:
        cp.wait()

    for i in range(_BPW // _L):
        v = idx_v[pl.ds(i * _L, _L)]
        idx_v[pl.ds(i * _L, _L)] = jnp.where(
            v < jnp.int32(0), jnp.int32(N_ROWS - 1), v)

    def fire_gather(c):
        return pltpu.async_copy(
            table_hbm.at[idx_v.at[pl.ds(c * _RCH, _RCH)]],
            rows_v.at[c % _NB], sem_g)

    gat = {0: fire_gather(0)}
    scat = {}
    for c in range(_NRCH):
        nxt = c + 1
        if nxt < _NRCH:
            if nxt >= _NB:
                scat[nxt - _NB].wait()
            gat[nxt] = fire_gather(nxt)
        gat[c].wait()
        scat[c] = pltpu.async_copy(
            rows_v.at[c % _NB], out_hbm.at[pl.ds(base + c * _RCH, _RCH)],
            sem_s)
    for c in range(_NRCH - _NB, _NRCH):
        scat[c].wait()


_sc_call = pl.kernel(
    _body,
    out_type=jax.ShapeDtypeStruct((B, DW), jnp.int32),
    mesh=plsc.VectorSubcoreMesh(core_axis_name="c", subcore_axis_name="s"),
    compiler_params=pltpu.CompilerParams(needs_layout_passes=False),
    scratch_types=[
        pltpu.VMEM((_BPW // 2,), jnp.int32),
        pltpu.VMEM((_BPW,), jnp.int32),
        pltpu.VMEM((_BPW,), jnp.int32),
        pltpu.VMEM((_NB, _RCH, DW), jnp.int32),
        pltpu.SemaphoreType.DMA,
        pltpu.SemaphoreType.DMA,
        pltpu.SemaphoreType.DMA,
    ],
)


def kernel(sigma, table, lut):
    sigma32 = lax.bitcast_convert_type(sigma.reshape(B // 2, 2), jnp.int32)
    table32 = lax.bitcast_convert_type(table.reshape(N_ROWS, DW, 2), jnp.int32)
    out32 = _sc_call(sigma32, table32, lut)
    return lax.bitcast_convert_type(out32, jnp.bfloat16).reshape(B, D)


# speedup vs baseline: 2.3145x; 2.3145x over previous
import jax
import jax.numpy as jnp
from jax import lax
from jax.experimental import pallas as pl
from jax.experimental.pallas import tpu as pltpu
from jax.experimental.pallas import tpu_sc as plsc

N_ROWS = 50
D = 1024
B = 16384

_info = plsc.get_sparse_core_info()
_NC, _NS, _L = _info.num_cores, _info.num_subcores, _info.num_lanes
_NW = _NC * _NS
_BPW = B // _NW
_PPW = _BPW // 2
_CH = 128
_NCH = _BPW // _CH
_RCH = 16
_NRCH = _PPW // _RCH
_NB = 3


def _body(sigma_hbm, tlow_hbm, lut_hbm, out_bf16_hbm, sigma_v,
          bits_v, idx_v, ra_v, rb_v, sem_lut, sem_a, sem_b, sem_s):
    out_hbm = out_bf16_hbm.bitcast(jnp.int32)

    wid = lax.axis_index("s") * _NC + lax.axis_index("c")
    base = pl.multiple_of(wid * _PPW, _PPW)

    pltpu.sync_copy(
        sigma_hbm.at[pl.ds(pl.multiple_of(wid * _PPW, _PPW), _PPW)], sigma_v)

    for i in range(_PPW // _L):
        w = sigma_v[pl.ds(i * _L, _L)]
        bits_v[pl.ds(i * _L, _L)] = lax.bitwise_and(w, jnp.int32(0xFFFF))
        bits_v[pl.ds(_PPW + i * _L, _L)] = lax.shift_right_logical(
            w, jnp.int32(16))

    lut_cps = [
        pltpu.async_copy(lut_hbm.at[bits_v.at[pl.ds(c * _CH, _CH)]],
                         idx_v.at[pl.ds(c * _CH, _CH)], sem_lut)
        for c in range(_NCH)
    ]
    for cp in lut_cps:
        cp.wait()

    for i in range(_BPW // _L):
        v = idx_v[pl.ds(i * _L, _L)]
        idx_v[pl.ds(i * _L, _L)] = jnp.where(
            v < jnp.int32(0), jnp.int32(N_ROWS - 1), v)

    def fire_even(c):
        return pltpu.async_copy(
            tlow_hbm.at[idx_v.at[pl.ds(c * _RCH, _RCH)]],
            ra_v.at[c % _NB], sem_a)

    def fire_odd(c):
        return pltpu.async_copy(
            tlow_hbm.at[idx_v.at[pl.ds(_PPW + c * _RCH, _RCH)]],
            rb_v.at[c % _NB], sem_b)

    def combine(b):
        av, bv = ra_v.at[b], rb_v.at[b]

        def row(r, _):
            def grp(i, _):
                a = av[r, pl.ds(i * _L, _L)]
                o = bv[r, pl.ds(i * _L, _L)]
                av[r, pl.ds(i * _L, _L)] = lax.bitwise_or(
                    a, lax.shift_left(o, jnp.int32(16)))
                return 0

            return lax.fori_loop(0, D // _L, grp, 0, unroll=2)

        lax.fori_loop(0, _RCH, row, 0)

    gat = {0: (fire_even(0), fire_odd(0))}
    scat = {}
    for c in range(_NRCH):
        nxt = c + 1
        if nxt < _NRCH:
            if nxt >= _NB:
                scat[nxt - _NB].wait()
            gat[nxt] = (fire_even(nxt), fire_odd(nxt))
        gat[c][0].wait()
        gat[c][1].wait()
        combine(c % _NB)
        scat[c] = pltpu.async_copy(
            ra_v.at[c % _NB], out_hbm.at[pl.ds(base + c * _RCH, _RCH)],
            sem_s)
    for c in range(_NRCH - _NB, _NRCH):
        scat[c].wait()


_sc_call = pl.kernel(
    _body,
    out_type=jax.ShapeDtypeStruct((B, D), jnp.bfloat16),
    mesh=plsc.VectorSubcoreMesh(core_axis_name="c", subcore_axis_name="s"),
    compiler_params=pltpu.CompilerParams(needs_layout_passes=False),
    scratch_types=[
        pltpu.VMEM((_PPW,), jnp.int32),
        pltpu.VMEM((_BPW,), jnp.int32),
        pltpu.VMEM((_BPW,), jnp.int32),
        pltpu.VMEM((_NB, _RCH, D), jnp.int32),
        pltpu.VMEM((_NB, _RCH, D), jnp.int32),
        pltpu.SemaphoreType.DMA,
        pltpu.SemaphoreType.DMA,
        pltpu.SemaphoreType.DMA,
        pltpu.SemaphoreType.DMA,
    ],
)


def kernel(sigma, table, lut):
    sigma32 = lax.bitcast_convert_type(sigma.reshape(B // 2, 2), jnp.int32)
    tlow = lax.bitcast_convert_type(table, jnp.uint16).astype(jnp.int32)
    return _sc_call(sigma32, tlow, lut)


# speedup vs baseline: 2.4659x; 1.0654x over previous
import jax
import jax.numpy as jnp
from jax import lax
from jax.experimental import pallas as pl
from jax.experimental.pallas import tpu as pltpu
from jax.experimental.pallas import tpu_sc as plsc

N_ROWS = 50
D = 1024
B = 16384

_info = plsc.get_sparse_core_info()
_NC, _NS, _L = _info.num_cores, _info.num_subcores, _info.num_lanes
_NW = _NC * _NS
_BPW = B // _NW
_PPW = _BPW // 2
_CH = 128
_NCH = _BPW // _CH
_RCH = 16
_NRCH = _PPW // _RCH
_NB = 3


def _body(sigma_hbm, tlow_hbm, lut_hbm, out_bf16_hbm, sigma_v,
          bits_v, idx_v, ra_v, tlow_v, sem_lut, sem_a, sem_s):
    out_hbm = out_bf16_hbm.bitcast(jnp.int32)

    wid = lax.axis_index("s") * _NC + lax.axis_index("c")
    base = pl.multiple_of(wid * _PPW, _PPW)

    stage_cp = pltpu.async_copy(tlow_hbm, tlow_v, sem_a)

    pltpu.sync_copy(
        sigma_hbm.at[pl.ds(pl.multiple_of(wid * _PPW, _PPW), _PPW)], sigma_v)

    for i in range(_PPW // _L):
        w = sigma_v[pl.ds(i * _L, _L)]
        bits_v[pl.ds(i * _L, _L)] = lax.bitwise_and(w, jnp.int32(0xFFFF))
        bits_v[pl.ds(_PPW + i * _L, _L)] = lax.shift_right_logical(
            w, jnp.int32(16))

    lut_cps = [
        pltpu.async_copy(lut_hbm.at[bits_v.at[pl.ds(c * _CH, _CH)]],
                         idx_v.at[pl.ds(c * _CH, _CH)], sem_lut)
        for c in range(_NCH)
    ]
    for cp in lut_cps:
        cp.wait()

    for i in range(_BPW // _L):
        v = idx_v[pl.ds(i * _L, _L)]
        idx_v[pl.ds(i * _L, _L)] = jnp.where(
            v < jnp.int32(0), jnp.int32(N_ROWS - 1), v)

    stage_cp.wait()

    def build_chunk(c, b):
        ev = idx_v[pl.ds(c * _RCH, _RCH)]
        ov = idx_v[pl.ds(_PPW + c * _RCH, _RCH)]

        dnums = lax.GatherDimensionNumbers(
            offset_dims=(), collapsed_slice_dims=(0,), start_index_map=(0,))

        def row(r, _):
            rb = jnp.full((_L, 1), r, dtype=jnp.int32)
            e = lax.gather(ev, rb, dnums, (1,),
                           mode=lax.GatherScatterMode.PROMISE_IN_BOUNDS)[0]
            o = lax.gather(ov, rb, dnums, (1,),
                           mode=lax.GatherScatterMode.PROMISE_IN_BOUNDS)[0]

            def grp(i, _):
                a = tlow_v[e, pl.ds(i * _L, _L)]
                h = tlow_v[o, pl.ds(i * _L, _L)]
                ra_v[b, r, pl.ds(i * _L, _L)] = lax.bitwise_or(
                    a, lax.shift_left(h, jnp.int32(16)))
                return 0

            lax.fori_loop(0, D // _L, grp, 0, unroll=4)
            return 0

        lax.fori_loop(0, _RCH, row, 0)

    scat = {}
    for c in range(_NRCH):
        b = c % _NB
        if c >= _NB:
            scat[c - _NB].wait()
        build_chunk(c, b)
        scat[c] = pltpu.async_copy(
            ra_v.at[b], out_hbm.at[pl.ds(base + c * _RCH, _RCH)], sem_s)
    for c in range(_NRCH - _NB, _NRCH):
        scat[c].wait()


_sc_call = pl.kernel(
    _body,
    out_type=jax.ShapeDtypeStruct((B, D), jnp.bfloat16),
    mesh=plsc.VectorSubcoreMesh(core_axis_name="c", subcore_axis_name="s"),
    compiler_params=pltpu.CompilerParams(needs_layout_passes=False),
    scratch_types=[
        pltpu.VMEM((_PPW,), jnp.int32),
        pltpu.VMEM((_BPW,), jnp.int32),
        pltpu.VMEM((_BPW,), jnp.int32),
        pltpu.VMEM((_NB, _RCH, D), jnp.int32),
        pltpu.VMEM((N_ROWS, D), jnp.int32),
        pltpu.SemaphoreType.DMA,
        pltpu.SemaphoreType.DMA,
        pltpu.SemaphoreType.DMA,
    ],
)


def kernel(sigma, table, lut):
    sigma32 = lax.bitcast_convert_type(sigma.reshape(B // 2, 2), jnp.int32)
    tlow = lax.bitcast_convert_type(table, jnp.uint16).astype(jnp.int32)
    return _sc_call(sigma32, tlow, lut)


# speedup vs baseline: 3.8765x; 1.5720x over previous
import jax
import jax.numpy as jnp
from jax import lax
from jax.experimental import pallas as pl
from jax.experimental.pallas import tpu as pltpu
from jax.experimental.pallas import tpu_sc as plsc

N_ROWS = 50
D = 1024
B = 16384

_info = plsc.get_sparse_core_info()
_NC, _NS, _L = _info.num_cores, _info.num_subcores, _info.num_lanes
_NW = _NC * _NS
_BPW = B // _NW
_PPW = _BPW // 2
_CH = 128
_NCH = _BPW // _CH
_RCH = 16
_NRCH = _PPW // _RCH
_NB = 3


def _body(sigma_hbm, tlow_hbm, lut_hbm, out_bf16_hbm, sigma_v,
          bits_v, idx_v, ra_v, tlow_v, sem_lut, sem_a, sem_s):
    out_hbm = out_bf16_hbm.bitcast(jnp.int32)

    wid = lax.axis_index("s") * _NC + lax.axis_index("c")
    base = pl.multiple_of(wid * _PPW, _PPW)

    stage_cp = pltpu.async_copy(tlow_hbm, tlow_v, sem_a)

    pltpu.sync_copy(
        sigma_hbm.at[pl.ds(pl.multiple_of(wid * _PPW, _PPW), _PPW)], sigma_v)

    for i in range(_PPW // _L):
        w = sigma_v[pl.ds(i * _L, _L)]
        bits_v[pl.ds(i * _L, _L)] = lax.bitwise_and(w, jnp.int32(0xFFFF))
        bits_v[pl.ds(_PPW + i * _L, _L)] = lax.shift_right_logical(
            w, jnp.int32(16))

    lut_cps = [
        pltpu.async_copy(lut_hbm.at[bits_v.at[pl.ds(c * _CH, _CH)]],
                         idx_v.at[pl.ds(c * _CH, _CH)], sem_lut)
        for c in range(_NCH)
    ]
    for cp in lut_cps:
        cp.wait()

    for i in range(_BPW // _L):
        v = idx_v[pl.ds(i * _L, _L)]
        idx_v[pl.ds(i * _L, _L)] = jnp.where(
            v < jnp.int32(0), jnp.int32(N_ROWS - 1), v)

    stage_cp.wait()

    def build_chunk(c, b):
        ev = idx_v[pl.ds(c * _RCH, _RCH)]
        ov = idx_v[pl.ds(_PPW + c * _RCH, _RCH)]

        dnums = lax.GatherDimensionNumbers(
            offset_dims=(), collapsed_slice_dims=(0,), start_index_map=(0,))

        @plsc.parallel_loop(0, _RCH)
        def _row(r):
            rb = jnp.full((_L, 1), r, dtype=jnp.int32)
            e = lax.gather(ev, rb, dnums, (1,),
                           mode=lax.GatherScatterMode.PROMISE_IN_BOUNDS)[0]
            o = lax.gather(ov, rb, dnums, (1,),
                           mode=lax.GatherScatterMode.PROMISE_IN_BOUNDS)[0]

            @plsc.parallel_loop(0, D // _L, unroll=4)
            def _grp(i):
                a = tlow_v[e, pl.ds(i * _L, _L)]
                h = tlow_v[o, pl.ds(i * _L, _L)]
                ra_v[b, r, pl.ds(i * _L, _L)] = lax.bitwise_or(
                    a, lax.shift_left(h, jnp.int32(16)))

    scat = {}
    for c in range(_NRCH):
        b = c % _NB
        if c >= _NB:
            scat[c - _NB].wait()
        build_chunk(c, b)
        scat[c] = pltpu.async_copy(
            ra_v.at[b], out_hbm.at[pl.ds(base + c * _RCH, _RCH)], sem_s)
    for c in range(_NRCH - _NB, _NRCH):
        scat[c].wait()


_sc_call = pl.kernel(
    _body,
    out_type=jax.ShapeDtypeStruct((B, D), jnp.bfloat16),
    mesh=plsc.VectorSubcoreMesh(core_axis_name="c", subcore_axis_name="s"),
    compiler_params=pltpu.CompilerParams(needs_layout_passes=False),
    scratch_types=[
        pltpu.VMEM((_PPW,), jnp.int32),
        pltpu.VMEM((_BPW,), jnp.int32),
        pltpu.VMEM((_BPW,), jnp.int32),
        pltpu.VMEM((_NB, _RCH, D), jnp.int32),
        pltpu.VMEM((N_ROWS, D), jnp.int32),
        pltpu.SemaphoreType.DMA,
        pltpu.SemaphoreType.DMA,
        pltpu.SemaphoreType.DMA,
    ],
)


def kernel(sigma, table, lut):
    sigma32 = lax.bitcast_convert_type(sigma.reshape(B // 2, 2), jnp.int32)
    tlow = lax.bitcast_convert_type(table, jnp.uint16).astype(jnp.int32)
    return _sc_call(sigma32, tlow, lut)


# speedup vs baseline: 4.1417x; 1.0684x over previous
import jax
import jax.numpy as jnp
from jax import lax
from jax.experimental import pallas as pl
from jax.experimental.pallas import tpu as pltpu
from jax.experimental.pallas import tpu_sc as plsc

N_ROWS = 50
D = 1024
B = 16384

_info = plsc.get_sparse_core_info()
_NC, _NS, _L = _info.num_cores, _info.num_subcores, _info.num_lanes
_NW = _NC * _NS
_BPW = B // _NW
_PPW = _BPW // 2
_CH = 128
_NCH = _BPW // _CH
_RCH = 16
_NRCH = _PPW // _RCH
_NB = 3


def _body(sigma_hbm, tlow_hbm, lut_hbm, out_bf16_hbm, sigma_v,
          bits_v, idx_v, ra_v, tlow_v, sem_lut, sem_a, sem_s):
    out_hbm = out_bf16_hbm.bitcast(jnp.int32)

    wid = lax.axis_index("s") * _NC + lax.axis_index("c")
    base = pl.multiple_of(wid * _PPW, _PPW)

    stage_cp = pltpu.async_copy(tlow_hbm, tlow_v, sem_a)

    pltpu.sync_copy(
        sigma_hbm.at[pl.ds(pl.multiple_of(wid * _PPW, _PPW), _PPW)], sigma_v)

    for i in range(_PPW // _L):
        w = sigma_v[pl.ds(i * _L, _L)]
        bits_v[pl.ds(i * _L, _L)] = lax.bitwise_and(w, jnp.int32(0xFFFF))
        bits_v[pl.ds(_PPW + i * _L, _L)] = lax.shift_right_logical(
            w, jnp.int32(16))

    lut_cps = [
        pltpu.async_copy(lut_hbm.at[bits_v.at[pl.ds(c * _CH, _CH)]],
                         idx_v.at[pl.ds(c * _CH, _CH)], sem_lut)
        for c in range(_NCH)
    ]
    for cp in lut_cps:
        cp.wait()

    for i in range(_BPW // _L):
        v = idx_v[pl.ds(i * _L, _L)]
        idx_v[pl.ds(i * _L, _L)] = jnp.where(
            v < jnp.int32(0), jnp.int32(N_ROWS - 1), v)

    stage_cp.wait()

    def build_chunk(c, b):
        ev = idx_v[pl.ds(c * _RCH, _RCH)]
        ov = idx_v[pl.ds(_PPW + c * _RCH, _RCH)]

        dnums = lax.GatherDimensionNumbers(
            offset_dims=(), collapsed_slice_dims=(0,), start_index_map=(0,))

        @plsc.parallel_loop(0, _RCH)
        def _row(r):
            rb = jnp.full((_L, 1), r, dtype=jnp.int32)
            e = lax.gather(ev, rb, dnums, (1,),
                           mode=lax.GatherScatterMode.PROMISE_IN_BOUNDS)[0]
            o = lax.gather(ov, rb, dnums, (1,),
                           mode=lax.GatherScatterMode.PROMISE_IN_BOUNDS)[0]
            eoff = e * jnp.int32(D)
            ooff = o * jnp.int32(D)

            @plsc.parallel_loop(0, D, step=_L, unroll=8)
            def _grp(i):
                a = tlow_v[pl.ds(eoff + i, _L)]
                h = tlow_v[pl.ds(ooff + i, _L)]
                ra_v[b, r, pl.ds(i, _L)] = lax.bitwise_or(
                    a, lax.shift_left(h, jnp.int32(16)))

    scat = {}
    for c in range(_NRCH):
        b = c % _NB
        if c >= _NB:
            scat[c - _NB].wait()
        build_chunk(c, b)
        scat[c] = pltpu.async_copy(
            ra_v.at[b], out_hbm.at[pl.ds(base + c * _RCH, _RCH)], sem_s)
    for c in range(_NRCH - _NB, _NRCH):
        scat[c].wait()


_sc_call = pl.kernel(
    _body,
    out_type=jax.ShapeDtypeStruct((B, D), jnp.bfloat16),
    mesh=plsc.VectorSubcoreMesh(core_axis_name="c", subcore_axis_name="s"),
    compiler_params=pltpu.CompilerParams(needs_layout_passes=False),
    scratch_types=[
        pltpu.VMEM((_PPW,), jnp.int32),
        pltpu.VMEM((_BPW,), jnp.int32),
        pltpu.VMEM((_BPW,), jnp.int32),
        pltpu.VMEM((_NB, _RCH, D), jnp.int32),
        pltpu.VMEM((N_ROWS * D,), jnp.int32),
        pltpu.SemaphoreType.DMA,
        pltpu.SemaphoreType.DMA,
        pltpu.SemaphoreType.DMA,
    ],
)


def kernel(sigma, table, lut):
    sigma32 = lax.bitcast_convert_type(sigma.reshape(B // 2, 2), jnp.int32)
    tlow = lax.bitcast_convert_type(table, jnp.uint16).astype(
        jnp.int32).reshape(N_ROWS * D)
    return _sc_call(sigma32, tlow, lut)
